# Initial kernel scaffold; baseline (speedup 1.0000x reference)
import jax
import jax.numpy as jnp
from jax.experimental import pallas as pl


def kernel(inputs, table):
    raise NotImplementedError("write your pallas kernel here")


# speedup vs baseline: 3.1893x; 3.1893x over previous
import functools

import jax
import jax.numpy as jnp
from jax import lax
from jax.experimental import pallas as pl
from jax.experimental.pallas import tpu as pltpu
from jax.experimental.pallas import tpu_sc as plsc

ROWS = 4096 * 200
D = 64
NC, NS = 2, 16
NW = NC * NS
PER_W = ROWS // NW
C = 128
NCHUNK = PER_W // C

_mesh = plsc.VectorSubcoreMesh(core_axis_name="c", subcore_axis_name="s")


@functools.partial(
    pl.kernel,
    mesh=_mesh,
    out_type=jax.ShapeDtypeStruct((ROWS, D), jnp.float32),
    scratch_types=[
        pltpu.VMEM((C,), jnp.int32),
        pltpu.VMEM((C, D), jnp.float32),
        pltpu.SemaphoreType.DMA,
    ],
    compiler_params=pltpu.CompilerParams(use_tc_tiling_on_sc=False),
)
def _emb_lookup(idx_hbm, table_hbm, out_hbm, idx_v, rows_v, sem):
    wid = lax.axis_index("s") * NC + lax.axis_index("c")
    base = wid * PER_W

    def body(j, carry):
        off = base + j * C
        pltpu.sync_copy(idx_hbm.at[pl.ds(off, C)], idx_v)
        pltpu.async_copy(table_hbm.at[idx_v], rows_v, sem).wait()
        pltpu.sync_copy(rows_v, out_hbm.at[pl.ds(off, C)])
        return carry

    lax.fori_loop(0, NCHUNK, body, 0)


def kernel(inputs, table):
    idx = inputs.reshape(-1).astype(jnp.int32)
    out = _emb_lookup(idx, table)
    return out.reshape(inputs.shape + (table.shape[1],))


# speedup vs baseline: 3.5215x; 1.1042x over previous
import functools

import jax
import jax.numpy as jnp
from jax import lax
from jax.experimental import pallas as pl
from jax.experimental.pallas import tpu as pltpu
from jax.experimental.pallas import tpu_sc as plsc

ROWS = 4096 * 200
D = 64
NC, NS = 2, 16
NW = NC * NS
PER_W = ROWS // NW
C = 128
NCHUNK = PER_W // C

_mesh = plsc.VectorSubcoreMesh(core_axis_name="c", subcore_axis_name="s")


@functools.partial(
    pl.kernel,
    mesh=_mesh,
    out_type=jax.ShapeDtypeStruct((ROWS, D), jnp.float32),
    scratch_types=[
        pltpu.VMEM((NCHUNK, C), jnp.int32),
        pltpu.VMEM((2, C, D), jnp.float32),
        pltpu.SemaphoreType.DMA,
        pltpu.SemaphoreType.DMA,
        pltpu.SemaphoreType.DMA,
        pltpu.SemaphoreType.DMA,
        pltpu.SemaphoreType.DMA,
    ],
    compiler_params=pltpu.CompilerParams(use_tc_tiling_on_sc=False),
)
def _emb_lookup(idx_hbm, table_hbm, out_hbm, idx_v, rows_v, sem_i,
                sem_g0, sem_g1, sem_o0, sem_o1):
    wid = lax.axis_index("s") * NC + lax.axis_index("c")
    base = wid * PER_W
    sems_g = (sem_g0, sem_g1)
    sems_o = (sem_o0, sem_o1)

    pltpu.async_copy(idx_hbm.at[wid], idx_v, sem_i).wait()

    def g_start(j, b):
        pltpu.async_copy(table_hbm.at[idx_v.at[j]], rows_v.at[b], sems_g[b])

    def g_wait(b):
        pltpu.make_async_copy(
            table_hbm.at[idx_v.at[0]], rows_v.at[b], sems_g[b]).wait()

    def o_start(j, b):
        pltpu.async_copy(rows_v.at[b], out_hbm.at[pl.ds(base + j * C, C)],
                         sems_o[b])

    def o_wait(b):
        pltpu.make_async_copy(
            rows_v.at[b], out_hbm.at[pl.ds(base, C)], sems_o[b]).wait()

    g_start(0, 0)

    @pl.loop(0, NCHUNK, step=2)
    def _pipeline(j0):
        g_wait(0)

        @pl.when(j0 > 0)
        def _():
            o_wait(1)

        g_start(j0 + 1, 1)
        o_start(j0, 0)

        g_wait(1)
        o_wait(0)

        @pl.when(j0 + 2 < NCHUNK)
        def _():
            g_start(j0 + 2, 0)

        o_start(j0 + 1, 1)

    o_wait(1)


def kernel(inputs, table):
    idx = inputs.reshape(NW, NCHUNK, C).astype(jnp.int32)
    out = _emb_lookup(idx, table)
    return out.reshape(inputs.shape + (table.shape[1],))


# speedup vs baseline: 4.2261x; 1.2001x over previous
import functools

import jax
import jax.numpy as jnp
from jax import lax
from jax.experimental import pallas as pl
from jax.experimental.pallas import tpu as pltpu
from jax.experimental.pallas import tpu_sc as plsc

SENT = 4096
W = 200
D = 64
VOC = 1000
NC, NS = 2, 16
NW = NC * NS
SB = SENT // NW

_mesh = plsc.VectorSubcoreMesh(core_axis_name="c", subcore_axis_name="s")


@functools.partial(
    pl.kernel,
    mesh=_mesh,
    out_type=jax.ShapeDtypeStruct((W, D // 8, NW, 8, SB), jnp.float32),
    scratch_types=[
        pltpu.VMEM((VOC * D,), jnp.float32),
        pltpu.VMEM((SB, W), jnp.int32),
        pltpu.VMEM((2, D, SB), jnp.float32),
        pltpu.SemaphoreType.DMA,
        pltpu.SemaphoreType.DMA,
        pltpu.SemaphoreType.DMA,
    ],
    compiler_params=pltpu.CompilerParams(use_tc_tiling_on_sc=False,
                                         needs_layout_passes=False),
)
def _emb_lookup(idx_hbm, table_hbm, out_hbm, table_v, idx_v, blk_v,
                sem_i, sem_o0, sem_o1):
    wid = lax.axis_index("s") * NC + lax.axis_index("c")
    sems_o = (sem_o0, sem_o1)

    pltpu.async_copy(table_hbm, table_v, sem_i).wait()
    pltpu.async_copy(idx_hbm.at[pl.ds(wid * SB, SB)], idx_v, sem_i).wait()

    lane = lax.iota(jnp.int32, 16)

    def build(w, b):
        wvec = jnp.zeros((16,), jnp.int32) + w
        for g in range(SB // 16):
            svec = lane + (g * 16)
            sidx = plsc.load_gather(idx_v, [svec, wvec])
            addr_g = sidx * D

            @plsc.parallel_loop(0, D, unroll=8)
            def _d(d, addr_g=addr_g, g=g):
                val = plsc.load_gather(table_v, [addr_g + d])
                blk_v[b, d, pl.ds(g * 16, 16)] = val

    def flush(w, b):
        for d8 in range(D // 8):
            pltpu.async_copy(blk_v.at[b, pl.ds(d8 * 8, 8)],
                             out_hbm.at[w, d8, wid], sems_o[b])

    def drain(b):
        for d8 in range(D // 8):
            pltpu.make_async_copy(blk_v.at[b, pl.ds(d8 * 8, 8)],
                                  out_hbm.at[0, d8, wid], sems_o[b]).wait()

    build(0, 0)
    flush(0, 0)

    @pl.loop(1, W - 1, step=2)
    def _pipeline(w0):
        build(w0, 1)
        drain(0)
        flush(w0, 1)
        build(w0 + 1, 0)
        drain(1)
        flush(w0 + 1, 0)

    build(W - 1, 1)
    drain(0)
    flush(W - 1, 1)
    drain(1)


def kernel(inputs, table):
    idx = inputs.astype(jnp.int32)
    out5 = _emb_lookup(idx, table.reshape(-1))
    return out5.transpose(2, 4, 0, 1, 3).reshape(SENT, W, D)


# speedup vs baseline: 20.3950x; 4.8260x over previous
import functools

import jax
import jax.numpy as jnp
from jax import lax
from jax.experimental import pallas as pl
from jax.experimental.pallas import tpu as pltpu
from jax.experimental.pallas import tpu_sc as plsc

SENT = 4096
W = 200
D = 64
VOC = 1000
NC, NS = 2, 16
NW = NC * NS
SB = SENT // NW

_mesh = plsc.VectorSubcoreMesh(core_axis_name="c", subcore_axis_name="s")


@functools.partial(
    pl.kernel,
    mesh=_mesh,
    out_type=jax.ShapeDtypeStruct((W, D // 8, NW, 8, SB), jnp.float32),
    scratch_types=[
        pltpu.VMEM((VOC * D,), jnp.float32),
        pltpu.VMEM((SB, W), jnp.int32),
        pltpu.VMEM((2, D, SB), jnp.float32),
        pltpu.SemaphoreType.DMA,
        pltpu.SemaphoreType.DMA,
        pltpu.SemaphoreType.DMA,
    ],
    compiler_params=pltpu.CompilerParams(use_tc_tiling_on_sc=False,
                                         needs_layout_passes=False),
)
def _emb_lookup(idx_hbm, table_hbm, out_hbm, table_v, idx_v, blk_v,
                sem_i, sem_o0, sem_o1):
    wid = lax.axis_index("s") * NC + lax.axis_index("c")
    sems_o = (sem_o0, sem_o1)

    pltpu.async_copy(table_hbm, table_v, sem_i).wait()
    pltpu.async_copy(idx_hbm.at[pl.ds(wid * SB, SB)], idx_v, sem_i).wait()

    lane = lax.iota(jnp.int32, 16)

    def build(w, b):
        wvec = jnp.zeros((16,), jnp.int32) + w
        for g in range(SB // 16):
            svec = lane + (g * 16)
            sidx = plsc.load_gather(idx_v, [svec, wvec])

            @plsc.parallel_loop(0, D, unroll=8)
            def _d(d, sidx=sidx, g=g):
                val = plsc.load_gather(table_v, [sidx + d * VOC])
                blk_v[b, d, pl.ds(g * 16, 16)] = val

    def flush(w, b):
        for d8 in range(D // 8):
            pltpu.async_copy(blk_v.at[b, pl.ds(d8 * 8, 8)],
                             out_hbm.at[w, d8, wid], sems_o[b])

    def drain(b):
        for d8 in range(D // 8):
            pltpu.make_async_copy(blk_v.at[b, pl.ds(d8 * 8, 8)],
                                  out_hbm.at[0, d8, wid], sems_o[b]).wait()

    build(0, 0)
    flush(0, 0)

    @pl.loop(1, W - 1, step=2)
    def _pipeline(w0):
        build(w0, 1)
        drain(0)
        flush(w0, 1)
        build(w0 + 1, 0)
        drain(1)
        flush(w0 + 1, 0)

    build(W - 1, 1)
    drain(0)
    flush(W - 1, 1)
    drain(1)


def kernel(inputs, table):
    idx = inputs.astype(jnp.int32)
    out5 = _emb_lookup(idx, table.T.reshape(-1))
    return out5.transpose(2, 4, 0, 1, 3).reshape(SENT, W, D)
